# Initial kernel scaffold; baseline (speedup 1.0000x reference)
import jax
import jax.numpy as jnp
from jax.experimental import pallas as pl


def kernel(x, edge_index, W_gcn0, b_gcn0, W_gat0, att_src0, att_dst0, b_gat0, W_gcn1, b_gcn1, W_gat1, att_src1, att_dst1, b_gat1, W_gcn2, b_gcn2, W_gat2, att_src2, att_dst2, b_gat2):
    raise NotImplementedError("write your pallas kernel here")


# speedup vs baseline: 11.5078x; 11.5078x over previous
import functools

import jax
import jax.numpy as jnp
from jax import lax
from jax.experimental import pallas as pl
from jax.experimental.pallas import tpu as pltpu
from jax.experimental.pallas import tpu_sc as plsc

N_NODES = 10000
N_PAD = 10240
E_EDGES = 320000
E_TOT = E_EDGES + N_NODES
CH = 128
NCH = 81
NT = 32
E_PAD = NT * NCH * CH
ROWS_PER_TILE = N_PAD // 16
D = 128
HID = 128
H = 4
F32 = jnp.float32

_MESH = plsc.VectorSubcoreMesh(core_axis_name="c", subcore_axis_name="s")
_SC_PARAMS = pltpu.CompilerParams(needs_layout_passes=False,
                                  use_tc_tiling_on_sc=False)


def _wid_base(rows_per_tile):
  c = lax.axis_index("c")
  s = lax.axis_index("s")
  w = c * 16 + s
  return c, s, w, s * rows_per_tile


def _deg_body(d_hbm, zrow_hbm, deg_out, didx_v, ones_v, deg_sp):
  c, s, w, base = _wid_base(ROWS_PER_TILE)
  pltpu.sync_copy(d_hbm.at[w], didx_v)

  def mk_ones(i, _):
    ones_v[pl.ds(i * 16, 16)] = jnp.ones((16,), F32)
    return 0

  lax.fori_loop(0, CH // 16, mk_ones, 0)
  pltpu.sync_copy(zrow_hbm.at[pl.ds(base, ROWS_PER_TILE)],
                  deg_sp.at[pl.ds(base, ROWS_PER_TILE)])
  plsc.subcore_barrier()

  def chunk(ci, _):
    pltpu.sync_copy(ones_v, deg_sp.at[didx_v.at[ci]], add=True)
    return 0

  lax.fori_loop(0, NCH, chunk, 0)
  plsc.subcore_barrier()
  pltpu.sync_copy(deg_sp.at[pl.ds(base, ROWS_PER_TILE)],
                  deg_out.at[c, pl.ds(base, ROWS_PER_TILE)])


def _sc_degree(d_arr, zrow):
  return pl.kernel(
      _deg_body,
      out_type=jax.ShapeDtypeStruct((2, N_PAD), F32),
      mesh=_MESH,
      compiler_params=_SC_PARAMS,
      scratch_types=[
          pltpu.VMEM((NCH, CH), jnp.int32),
          pltpu.VMEM((CH,), F32),
          pltpu.VMEM_SHARED((N_PAD,), F32),
      ],
  )(d_arr, zrow)


def _segsum_body(tab_hbm, s_hbm, d_hbm, ztab_hbm, out_hbm,
                 sidx_v, didx_v, rowbuf, acc_sp, sem):
  c, s, w, base = _wid_base(ROWS_PER_TILE)
  pltpu.sync_copy(s_hbm.at[w], sidx_v)
  pltpu.sync_copy(d_hbm.at[w], didx_v)
  pltpu.sync_copy(ztab_hbm.at[pl.ds(base, ROWS_PER_TILE)],
                  acc_sp.at[pl.ds(base, ROWS_PER_TILE)])
  plsc.subcore_barrier()

  def chunk(ci, _):
    pltpu.async_copy(tab_hbm.at[sidx_v.at[ci]], rowbuf, sem).wait()
    pltpu.sync_copy(rowbuf, acc_sp.at[didx_v.at[ci]], add=True)
    return 0

  lax.fori_loop(0, NCH, chunk, 0)
  plsc.subcore_barrier()
  pltpu.sync_copy(acc_sp.at[pl.ds(base, ROWS_PER_TILE)],
                  out_hbm.at[c, pl.ds(base, ROWS_PER_TILE)])


def _sc_segsum(table, s_arr, d_arr, ztab):
  return pl.kernel(
      _segsum_body,
      out_type=jax.ShapeDtypeStruct((2, N_PAD, D), F32),
      mesh=_MESH,
      compiler_params=_SC_PARAMS,
      scratch_types=[
          pltpu.VMEM((NCH, CH), jnp.int32),
          pltpu.VMEM((NCH, CH), jnp.int32),
          pltpu.VMEM((CH, D), F32),
          pltpu.VMEM_SHARED((N_PAD, D), F32),
          pltpu.SemaphoreType.DMA,
      ],
  )(table, s_arr, d_arr, ztab)


def _lane_idx(k):
  lane = lax.iota(jnp.int32, 16)
  row = k * 4 + lax.shift_right_logical(lane, 2)
  col = lane & 3
  return row, col


def _den_body(als_hbm, ald_hbm, msv_hbm, s_hbm, d_hbm, zden_hbm,
              den_out, ex_out, sidx_v, didx_v, asbuf, adbuf, exbuf,
              msbuf, den_sp, sem):
  c, s, w, base = _wid_base(ROWS_PER_TILE)
  pltpu.sync_copy(s_hbm.at[w], sidx_v)
  pltpu.sync_copy(d_hbm.at[w], didx_v)
  pltpu.sync_copy(msv_hbm, msbuf)
  pltpu.sync_copy(zden_hbm.at[pl.ds(base, ROWS_PER_TILE)],
                  den_sp.at[pl.ds(base, ROWS_PER_TILE)])
  plsc.subcore_barrier()
  ms = msbuf[...]

  def chunk(ci, _):
    pltpu.async_copy(als_hbm.at[sidx_v.at[ci]], asbuf, sem).wait()
    pltpu.async_copy(ald_hbm.at[didx_v.at[ci]], adbuf, sem).wait()
    for k in range(CH * 4 // 16):
      row, col = _lane_idx(k)
      a_s = plsc.load_gather(asbuf, [row, col])
      a_d = plsc.load_gather(adbuf, [row, col])
      e = a_s + a_d
      e = jnp.where(e >= 0, e, 0.2 * e)
      bnd = ms + a_d
      bnd = jnp.where(bnd >= 0, bnd, 0.2 * bnd)
      ex = jnp.exp(e - bnd)
      plsc.store_scatter(exbuf, [row, col], ex)
    pltpu.sync_copy(exbuf, den_sp.at[didx_v.at[ci]], add=True)
    pltpu.sync_copy(exbuf, ex_out.at[w, ci])
    return 0

  lax.fori_loop(0, NCH, chunk, 0)
  plsc.subcore_barrier()
  pltpu.sync_copy(den_sp.at[pl.ds(base, ROWS_PER_TILE)],
                  den_out.at[c, pl.ds(base, ROWS_PER_TILE)])


def _sc_den(als, ald, msv, s_arr, d_arr, zden):
  return pl.kernel(
      _den_body,
      out_type=(
          jax.ShapeDtypeStruct((2, N_PAD, H), F32),
          jax.ShapeDtypeStruct((NT, NCH, CH, H), F32),
      ),
      mesh=_MESH,
      compiler_params=_SC_PARAMS,
      scratch_types=[
          pltpu.VMEM((NCH, CH), jnp.int32),
          pltpu.VMEM((NCH, CH), jnp.int32),
          pltpu.VMEM((CH, H), F32),
          pltpu.VMEM((CH, H), F32),
          pltpu.VMEM((CH, H), F32),
          pltpu.VMEM((16,), F32),
          pltpu.VMEM_SHARED((N_PAD, H), F32),
          pltpu.SemaphoreType.DMA,
      ],
  )(als, ald, msv, s_arr, d_arr, zden)


CH2 = 64
NCH2 = E_PAD // (NT * CH2)


def _wagg_body(h_hbm, ex_hbm, den0_hbm, den1_hbm, s_hbm, d_hbm, ztab_hbm,
               out_hbm, sidx_v, didx_v, idxbuf, rowbuf, exbuf, d0buf,
               d1buf, wbuf, outbuf, acc_sp, sem):
  c, s, w, base = _wid_base(ROWS_PER_TILE)
  pltpu.sync_copy(s_hbm.at[w], sidx_v)
  pltpu.sync_copy(d_hbm.at[w], didx_v)
  pltpu.sync_copy(ztab_hbm.at[pl.ds(base, ROWS_PER_TILE)],
                  acc_sp.at[pl.ds(base, ROWS_PER_TILE)])
  plsc.subcore_barrier()

  def chunk(ci, _):
    pltpu.sync_copy(ex_hbm.at[w, ci], exbuf)
    pltpu.async_copy(den0_hbm.at[didx_v.at[ci]], d0buf, sem).wait()
    pltpu.async_copy(den1_hbm.at[didx_v.at[ci]], d1buf, sem).wait()
    for k in range(CH2 * 4 // 16):
      row, col = _lane_idx(k)
      ex = plsc.load_gather(exbuf, [row, col])
      d0 = plsc.load_gather(d0buf, [row, col])
      d1 = plsc.load_gather(d1buf, [row, col])
      wgt = ex * 0.25 / (d0 + d1 + 1e-16)
      plsc.store_scatter(wbuf, [row, col], wgt)

    for hd in range(H):
      for j in range(CH2 // 16):
        idxbuf[pl.ds(j * 16, 16)] = (
            sidx_v[ci, pl.ds(j * 16, 16)] + hd * N_PAD)
      pltpu.async_copy(h_hbm.at[idxbuf], rowbuf, sem).wait()

      def one_row(r, _):
        wsp = plsc.load_gather(
            wbuf, [jnp.full((16,), 0, jnp.int32) + r,
                   jnp.full((16,), hd, jnp.int32)])
        for j in range(D // 16):
          v = wsp * rowbuf[r, pl.ds(j * 16, 16)]
          if hd == 0:
            outbuf[r, pl.ds(j * 16, 16)] = v
          else:
            outbuf[r, pl.ds(j * 16, 16)] += v
        return 0

      lax.fori_loop(0, CH2, one_row, 0)
    pltpu.sync_copy(outbuf, acc_sp.at[didx_v.at[ci]], add=True)
    return 0

  lax.fori_loop(0, NCH2, chunk, 0)
  plsc.subcore_barrier()
  pltpu.sync_copy(acc_sp.at[pl.ds(base, ROWS_PER_TILE)],
                  out_hbm.at[c, pl.ds(base, ROWS_PER_TILE)])


def _sc_wagg(h_flat, ex, den0, den1, s_arr, d_arr, ztab):
  return pl.kernel(
      _wagg_body,
      out_type=jax.ShapeDtypeStruct((2, N_PAD, D), F32),
      mesh=_MESH,
      compiler_params=_SC_PARAMS,
      scratch_types=[
          pltpu.VMEM((NCH2, CH2), jnp.int32),
          pltpu.VMEM((NCH2, CH2), jnp.int32),
          pltpu.VMEM((CH2,), jnp.int32),
          pltpu.VMEM((CH2, D), F32),
          pltpu.VMEM((CH2, H), F32),
          pltpu.VMEM((CH2, H), F32),
          pltpu.VMEM((CH2, H), F32),
          pltpu.VMEM((CH2, H), F32),
          pltpu.VMEM((CH2, D), F32),
          pltpu.VMEM_SHARED((N_PAD, D), F32),
          pltpu.SemaphoreType.DMA,
      ],
  )(h_flat, ex, den0, den1, s_arr, d_arr, ztab)


_RB = 512
_NRB = N_PAD // _RB


def _mm_scale_body(u_ref, dis_ref, w_ref, o_ref):
  o_ref[...] = jnp.dot(u_ref[...] * dis_ref[...], w_ref[...],
                       preferred_element_type=F32)


def _tc_mm_scaled(u, dis_b, W):
  return pl.pallas_call(
      _mm_scale_body,
      grid=(_NRB,),
      in_specs=[
          pl.BlockSpec((_RB, D), lambda i: (i, 0)),
          pl.BlockSpec((_RB, D), lambda i: (i, 0)),
          pl.BlockSpec((D, D), lambda i: (0, 0)),
      ],
      out_specs=pl.BlockSpec((_RB, D), lambda i: (i, 0)),
      out_shape=jax.ShapeDtypeStruct((N_PAD, D), F32),
  )(u, dis_b, W)


def _gat_dense_body(u_ref, w_ref, asrc_ref, adst_ref, h_ref, als_ref,
                    ald_ref, ms_ref, acc_ref):
  i = pl.program_id(0)
  h = jnp.dot(u_ref[...], w_ref[...], preferred_element_type=F32)
  for hd in range(H):
    h_ref[hd] = h[:, hd * D:(hd + 1) * D]
  als = jnp.dot(h, asrc_ref[...], preferred_element_type=F32)
  ald = jnp.dot(h, adst_ref[...], preferred_element_type=F32)
  als_ref[...] = als
  ald_ref[...] = ald
  blk_max = jnp.max(als.reshape(_RB // 8, 8, 128), axis=0)

  @pl.when(i == 0)
  def _():
    acc_ref[...] = blk_max

  @pl.when(i > 0)
  def _():
    acc_ref[...] = jnp.maximum(acc_ref[...], blk_max)

  ms_ref[...] = acc_ref[...]


def _tc_gat_dense(u, W, asrc_m, adst_m):
  return pl.pallas_call(
      _gat_dense_body,
      grid=(_NRB,),
      in_specs=[
          pl.BlockSpec((_RB, D), lambda i: (i, 0)),
          pl.BlockSpec((D, H * D), lambda i: (0, 0)),
          pl.BlockSpec((H * D, 128), lambda i: (0, 0)),
          pl.BlockSpec((H * D, 128), lambda i: (0, 0)),
      ],
      out_specs=[
          pl.BlockSpec((H, _RB, D), lambda i: (0, i, 0)),
          pl.BlockSpec((_RB, 128), lambda i: (i, 0)),
          pl.BlockSpec((_RB, 128), lambda i: (i, 0)),
          pl.BlockSpec((8, 128), lambda i: (0, 0)),
      ],
      out_shape=[
          jax.ShapeDtypeStruct((H, N_PAD, D), F32),
          jax.ShapeDtypeStruct((N_PAD, 128), F32),
          jax.ShapeDtypeStruct((N_PAD, 128), F32),
          jax.ShapeDtypeStruct((8, 128), F32),
      ],
      scratch_shapes=[pltpu.VMEM((8, 128), F32)],
  )(u, W, asrc_m, adst_m)


def _combine_body(p_ref, b_ref, res_ref, dis_ref, o_ref, *, scale_by_dis,
                  alpha, relu):
  agg = p_ref[0] + p_ref[1]
  if scale_by_dis:
    agg = agg * dis_ref[...]
  else:
    agg = agg * alpha
  y = agg + b_ref[0:1, :] + res_ref[...]
  if relu:
    y = jnp.maximum(y, 0.0)
  o_ref[...] = y


def _tc_combine(p2, b8, res, dis_b, scale_by_dis, alpha, relu):
  body = functools.partial(_combine_body, scale_by_dis=scale_by_dis,
                           alpha=alpha, relu=relu)
  return pl.pallas_call(
      body,
      grid=(_NRB,),
      in_specs=[
          pl.BlockSpec((2, _RB, D), lambda i: (0, i, 0)),
          pl.BlockSpec((8, D), lambda i: (0, 0)),
          pl.BlockSpec((_RB, D), lambda i: (i, 0)),
          pl.BlockSpec((_RB, D), lambda i: (i, 0)),
      ],
      out_specs=pl.BlockSpec((_RB, D), lambda i: (i, 0)),
      out_shape=jax.ShapeDtypeStruct((N_PAD, D), F32),
  )(p2, b8, res, dis_b)


def _gcn_layer(u, dis_b, W, b, s_arr, d_arr, ztab, res, relu):
  table = _tc_mm_scaled(u, dis_b, W)
  p2 = _sc_segsum(table, s_arr, d_arr, ztab)
  b8 = jnp.broadcast_to(b[None, :], (8, D))
  return _tc_combine(p2, b8, res, dis_b, True, 1.0, relu)


def _gat_layer(u, W, att_src, att_dst, b, s_arr, d_arr, ztab, zden, res,
               relu):
  eyeh = jnp.eye(H, 128, dtype=F32)
  asrc_m = (att_src[:, :, None] * eyeh[:, None, :]).reshape(H * D, 128)
  adst_m = (att_dst[:, :, None] * eyeh[:, None, :]).reshape(H * D, 128)
  h, als_pad, ald_pad, ms = _tc_gat_dense(u, W, asrc_m, adst_m)
  als = als_pad[:, :H]
  ald = ald_pad[:, :H]
  msv = jnp.tile(jnp.max(ms, axis=0)[:H], 4)
  den2, ex = _sc_den(als, ald, msv, s_arr, d_arr, zden)
  den0 = den2[0]
  den1 = den2[1]
  h_flat = h.reshape(H * N_PAD, D)
  s2 = s_arr.reshape(NT, NCH2, CH2)
  d2 = d_arr.reshape(NT, NCH2, CH2)
  ex2 = ex.reshape(NT, NCH2, CH2, H)
  p2 = _sc_wagg(h_flat, ex2, den0, den1, s2, d2, ztab)
  b8 = jnp.broadcast_to(b[None, :], (8, D))
  dummy_dis = ztab
  return _tc_combine(p2, b8, res, dummy_dis, False, 0.25, relu)


def kernel(x, edge_index, W_gcn0, b_gcn0, W_gat0, att_src0, att_dst0,
           b_gat0, W_gcn1, b_gcn1, W_gat1, att_src1, att_dst1, b_gat1,
           W_gcn2, b_gcn2, W_gat2, att_src2, att_dst2, b_gat2):
  n = N_NODES
  x_pad = jnp.zeros((N_PAD, D), F32).at[:n].set(x)

  loop = jnp.arange(n, dtype=jnp.int32)
  pad = jnp.full((E_PAD - E_TOT,), n, jnp.int32)
  s_arr = jnp.concatenate([edge_index[0], loop, pad]).reshape(NT, NCH, CH)
  d_arr = jnp.concatenate([edge_index[1], loop, pad]).reshape(NT, NCH, CH)

  zrow = jnp.zeros((N_PAD,), F32)
  ztab = jnp.zeros((N_PAD, D), F32)
  zden = jnp.zeros((N_PAD, H), F32)

  deg2 = _sc_degree(d_arr, zrow)
  deg = deg2[0] + deg2[1]
  dis = jnp.where(deg > 0, lax.rsqrt(deg), 0.0)
  dis_b = jnp.broadcast_to(dis[:, None], (N_PAD, D))

  zeros_res = jnp.zeros((N_PAD, D), F32)
  xg = _gcn_layer(x_pad, dis_b, W_gcn0, b_gcn0, s_arr, d_arr, ztab,
                  zeros_res, True)
  xg = _gcn_layer(xg, dis_b, W_gcn1, b_gcn1, s_arr, d_arr, ztab,
                  zeros_res, True)
  xg = _gcn_layer(xg, dis_b, W_gcn2, b_gcn2, s_arr, d_arr, ztab,
                  x_pad, False)

  xa = _gat_layer(x_pad, W_gat0, att_src0, att_dst0, b_gat0, s_arr, d_arr,
                  ztab, zden, zeros_res, True)
  xa = _gat_layer(xa, W_gat1, att_src1, att_dst1, b_gat1, s_arr, d_arr,
                  ztab, zden, zeros_res, True)
  xa = _gat_layer(xa, W_gat2, att_src2, att_dst2, b_gat2, s_arr, d_arr,
                  ztab, zden, x_pad, False)

  return jnp.concatenate([xg[:n], xa[:n]], axis=1)
